# baseline (speedup 1.0000x reference)
import functools

import jax
import jax.numpy as jnp
from jax import lax
from jax.experimental import pallas as pl
from jax.experimental.pallas import tpu as pltpu
from jax.experimental.pallas import tpu_sc as plsc

_NC = 2
_NS = 16
_NW = _NC * _NS


def _mask_body(k_keep, w1_ref, w2_ref, m1_ref, m2_ref, a_ref):
    r1 = w1_ref.shape[0]
    r2 = w2_ref.shape[0]
    a_ref[pl.ds(0, r1), :] = lax.bitcast_convert_type(
        jnp.abs(w1_ref[...]), jnp.int32)
    a_ref[pl.ds(r1, r2), :] = lax.bitcast_convert_type(
        jnp.abs(w2_ref[...]), jnp.int32)

    def body(_, carry):
        u, p = carry
        cand = u | p
        cnt = jnp.sum((a_ref[...] >= cand).astype(jnp.int32))
        u = jnp.where(cnt >= k_keep, cand, u)
        return u, p >> 1

    u, _ = lax.fori_loop(0, 31, body, (jnp.int32(0), jnp.int32(1) << 30))
    thr = lax.bitcast_convert_type(u, jnp.float32)
    w1 = w1_ref[...]
    w2 = w2_ref[...]
    m1_ref[...] = jnp.where(jnp.abs(w1) >= thr, w1, 0.0).astype(jnp.bfloat16)
    m2_ref[...] = jnp.where(jnp.abs(w2) >= thr, w2, 0.0).astype(jnp.bfloat16)


def _masked_weights(W1, W2):
    n = W1.size + W2.size
    k_keep = max(int(n * 0.5), 1)
    return pl.pallas_call(
        functools.partial(_mask_body, k_keep),
        out_shape=[
            jax.ShapeDtypeStruct(W1.shape, jnp.bfloat16),
            jax.ShapeDtypeStruct(W2.shape, jnp.bfloat16),
        ],
        scratch_shapes=[
            pltpu.VMEM((W1.shape[0] + W2.shape[0], W1.shape[1]), jnp.int32),
        ],
    )(W1, W2)


def _gather(emb, idx):
    n_rows = idx.shape[0]
    d = emb.shape[1]
    per_w = n_rows // _NW
    ch = 32
    n_ch = per_w // ch
    mesh = plsc.VectorSubcoreMesh(core_axis_name="c", subcore_axis_name="s")

    @functools.partial(
        pl.kernel,
        mesh=mesh,
        out_type=jax.ShapeDtypeStruct((n_rows, d), jnp.float32),
        scratch_types=[
            pltpu.VMEM((per_w,), jnp.int32),
            pltpu.VMEM((ch, d), jnp.float32),
            pltpu.VMEM((ch, d), jnp.float32),
            pltpu.VMEM((ch, d), jnp.float32),
            pltpu.SemaphoreType.DMA,
            pltpu.SemaphoreType.DMA,
            pltpu.SemaphoreType.DMA,
        ],
    )
    def k(table_hbm, idx_hbm, out_hbm, idx_v, rows_a, rows_b, rows_c,
          sem_a, sem_b, sem_c):
        wid = lax.axis_index("s") * _NC + lax.axis_index("c")
        base = wid * per_w
        pltpu.sync_copy(idx_hbm.at[pl.ds(base, per_w)], idx_v)
        bufs = (rows_a, rows_b, rows_c)
        sems = (sem_a, sem_b, sem_c)
        nb = len(bufs)
        for c in range(min(nb - 1, n_ch)):
            pltpu.async_copy(table_hbm.at[idx_v.at[pl.ds(c * ch, ch)]],
                             bufs[c], sems[c])
        for c in range(n_ch):
            pltpu.make_async_copy(
                table_hbm.at[idx_v.at[pl.ds(c * ch, ch)]], bufs[c % nb],
                sems[c % nb]).wait()
            if c + nb - 1 < n_ch:
                cn = c + nb - 1
                pltpu.async_copy(
                    table_hbm.at[idx_v.at[pl.ds(cn * ch, ch)]],
                    bufs[cn % nb], sems[cn % nb])
            pltpu.sync_copy(bufs[c % nb], out_hbm.at[pl.ds(base + c * ch, ch)])

    return k(emb, idx)


def _fwd_body(n_batch, m_blk, x_ref, dec_ref, w1_ref, w2_ref, len_ref, out_ref):
    i = pl.program_id(0)
    sub = 512
    n_sub = m_blk // sub
    w1 = w1_ref[...]
    w2 = w2_ref[...]
    for t in range(n_sub):
        x = x_ref[pl.ds(t * sub, sub), :]
        h = jnp.tanh(jnp.dot(x.astype(jnp.bfloat16), w1,
                             preferred_element_type=jnp.float32))
        row = (i * m_blk + t * sub
               + lax.broadcasted_iota(jnp.int32, (sub, 1), 0))
        s = row // n_batch
        b = row % n_batch
        lb = jnp.full_like(row, len_ref[0])
        for j in range(1, n_batch):
            lb = jnp.where(b == j, len_ref[j], lb)
        mask = (s < lb).astype(jnp.float32)
        z = (h * mask + dec_ref[pl.ds(t * sub, sub), :]).astype(jnp.bfloat16)
        out = jnp.dot(z, w2, preferred_element_type=jnp.float32)
        out_ref[pl.ds(t * (sub // n_batch), sub // n_batch), :, :] = (
            out.reshape(sub // n_batch, n_batch, out.shape[-1]))


def _forward(rows, W1m, W2m, lengths, n_seq, n_batch):
    sb = n_seq * n_batch
    d = W1m.shape[0]
    m_blk = 1024
    s_blk = m_blk // n_batch
    grid = (sb // m_blk,)
    dec_off = sb // m_blk
    return pl.pallas_call(
        functools.partial(_fwd_body, n_batch, m_blk),
        grid=grid,
        in_specs=[
            pl.BlockSpec((m_blk, d), lambda i: (i, 0)),
            pl.BlockSpec((m_blk, d), lambda i, o=dec_off: (i + o, 0)),
            pl.BlockSpec((d, d), lambda i: (0, 0)),
            pl.BlockSpec((d, d), lambda i: (0, 0)),
            pl.BlockSpec(memory_space=pltpu.SMEM),
        ],
        out_specs=pl.BlockSpec((s_blk, n_batch, d), lambda i: (i, 0, 0)),
        out_shape=jax.ShapeDtypeStruct((n_seq, n_batch, d), jnp.float32),
    )(rows, rows, W1m, W2m, lengths)


def kernel(src, tgt, lengths, emb, W1, W2):
    n_seq, n_batch = src.shape
    d = emb.shape[1]
    W1m, W2m = _masked_weights(W1, W2)
    idx = jnp.concatenate([src.reshape(-1), tgt.reshape(-1)]).astype(jnp.int32)
    rows = _gather(emb, idx)
    return _forward(rows, W1m, W2m, lengths.astype(jnp.int32), n_seq, n_batch)


# speedup vs baseline: 1.2265x; 1.2265x over previous
import functools

import jax
import jax.numpy as jnp
from jax import lax
from jax.experimental import pallas as pl
from jax.experimental.pallas import tpu as pltpu
from jax.experimental.pallas import tpu_sc as plsc

_NC = 2
_NS = 16
_NW = _NC * _NS


def _mask_body(k_keep, w1_ref, w2_ref, m1_ref, m2_ref, a1_ref, a2_ref):
    a1_ref[...] = lax.bitcast_convert_type(jnp.abs(w1_ref[...]), jnp.int32)
    a2_ref[...] = lax.bitcast_convert_type(jnp.abs(w2_ref[...]), jnp.int32)

    def body(_, carry):
        u, p = carry
        cand = u | p
        cnt = (jnp.sum((a1_ref[...] >= cand).astype(jnp.int32))
               + jnp.sum((a2_ref[...] >= cand).astype(jnp.int32)))
        u = jnp.where(cnt >= k_keep, cand, u)
        return u, p >> 1

    u, _ = lax.fori_loop(0, 31, body, (jnp.int32(0), jnp.int32(1) << 30))
    thr = lax.bitcast_convert_type(u, jnp.float32)
    w1 = w1_ref[...]
    w2 = w2_ref[...]
    m1_ref[...] = jnp.where(jnp.abs(w1) >= thr, w1, 0.0).astype(jnp.bfloat16)
    m2_ref[...] = jnp.where(jnp.abs(w2) >= thr, w2, 0.0).astype(jnp.bfloat16)


def _masked_weights(W1, W2):
    n = W1.size + W2.size
    k_keep = max(int(n * 0.5), 1)
    return pl.pallas_call(
        functools.partial(_mask_body, k_keep),
        out_shape=[
            jax.ShapeDtypeStruct(W1.shape, jnp.bfloat16),
            jax.ShapeDtypeStruct(W2.shape, jnp.bfloat16),
        ],
        scratch_shapes=[
            pltpu.VMEM(W1.shape, jnp.int32),
            pltpu.VMEM(W2.shape, jnp.int32),
        ],
    )(W1, W2)


def _gather(emb, idx):
    n_rows = idx.shape[0]
    d = emb.shape[1]
    per_w = n_rows // _NW
    ch = 32
    n_ch = per_w // ch
    mesh = plsc.VectorSubcoreMesh(core_axis_name="c", subcore_axis_name="s")

    @functools.partial(
        pl.kernel,
        mesh=mesh,
        out_type=jax.ShapeDtypeStruct((n_rows, d), jnp.float32),
        scratch_types=[
            pltpu.VMEM((per_w,), jnp.int32),
            pltpu.VMEM((ch, d), jnp.float32),
            pltpu.VMEM((ch, d), jnp.float32),
            pltpu.VMEM((ch, d), jnp.float32),
            pltpu.SemaphoreType.DMA,
            pltpu.SemaphoreType.DMA,
            pltpu.SemaphoreType.DMA,
        ],
    )
    def k(table_hbm, idx_hbm, out_hbm, idx_v, rows_a, rows_b, rows_c,
          sem_a, sem_b, sem_c):
        wid = lax.axis_index("s") * _NC + lax.axis_index("c")
        base = wid * per_w
        pltpu.sync_copy(idx_hbm.at[pl.ds(base, per_w)], idx_v)
        bufs = (rows_a, rows_b, rows_c)
        sems = (sem_a, sem_b, sem_c)
        nb = len(bufs)
        for c in range(min(nb - 1, n_ch)):
            pltpu.async_copy(table_hbm.at[idx_v.at[pl.ds(c * ch, ch)]],
                             bufs[c], sems[c])
        for c in range(n_ch):
            pltpu.make_async_copy(
                table_hbm.at[idx_v.at[pl.ds(c * ch, ch)]], bufs[c % nb],
                sems[c % nb]).wait()
            if c + nb - 1 < n_ch:
                cn = c + nb - 1
                pltpu.async_copy(
                    table_hbm.at[idx_v.at[pl.ds(cn * ch, ch)]],
                    bufs[cn % nb], sems[cn % nb])
            pltpu.sync_copy(bufs[c % nb], out_hbm.at[pl.ds(base + c * ch, ch)])

    return k(emb, idx)


def _fwd_body(n_batch, m_blk, x_ref, dec_ref, w1_ref, w2_ref, len_ref, out_ref):
    i = pl.program_id(0)
    sub = 512
    n_sub = m_blk // sub
    w1 = w1_ref[...]
    w2 = w2_ref[...]
    for t in range(n_sub):
        x = x_ref[pl.ds(t * sub, sub), :]
        h = jnp.tanh(jnp.dot(x.astype(jnp.bfloat16), w1,
                             preferred_element_type=jnp.float32))
        row = (i * m_blk + t * sub
               + lax.broadcasted_iota(jnp.int32, (sub, 1), 0))
        s = row // n_batch
        b = row % n_batch
        lb = jnp.full_like(row, len_ref[0])
        for j in range(1, n_batch):
            lb = jnp.where(b == j, len_ref[j], lb)
        mask = (s < lb).astype(jnp.float32)
        z = (h * mask + dec_ref[pl.ds(t * sub, sub), :]).astype(jnp.bfloat16)
        out = jnp.dot(z, w2, preferred_element_type=jnp.float32)
        out_ref[pl.ds(t * (sub // n_batch), sub // n_batch), :, :] = (
            out.reshape(sub // n_batch, n_batch, out.shape[-1]))


def _forward(rows, W1m, W2m, lengths, n_seq, n_batch):
    sb = n_seq * n_batch
    d = W1m.shape[0]
    m_blk = 1024
    s_blk = m_blk // n_batch
    grid = (sb // m_blk,)
    dec_off = sb // m_blk
    return pl.pallas_call(
        functools.partial(_fwd_body, n_batch, m_blk),
        grid=grid,
        in_specs=[
            pl.BlockSpec((m_blk, d), lambda i: (i, 0)),
            pl.BlockSpec((m_blk, d), lambda i, o=dec_off: (i + o, 0)),
            pl.BlockSpec((d, d), lambda i: (0, 0)),
            pl.BlockSpec((d, d), lambda i: (0, 0)),
            pl.BlockSpec(memory_space=pltpu.SMEM),
        ],
        out_specs=pl.BlockSpec((s_blk, n_batch, d), lambda i: (i, 0, 0)),
        out_shape=jax.ShapeDtypeStruct((n_seq, n_batch, d), jnp.float32),
    )(rows, rows, W1m, W2m, lengths)


def kernel(src, tgt, lengths, emb, W1, W2):
    n_seq, n_batch = src.shape
    d = emb.shape[1]
    W1m, W2m = _masked_weights(W1, W2)
    idx = jnp.concatenate([src.reshape(-1), tgt.reshape(-1)]).astype(jnp.int32)
    rows = _gather(emb, idx)
    return _forward(rows, W1m, W2m, lengths.astype(jnp.int32), n_seq, n_batch)
